# Initial kernel scaffold; baseline (speedup 1.0000x reference)
import jax
import jax.numpy as jnp
from jax.experimental import pallas as pl


def kernel(x, edge_index, W1, b1, W2, b2):
    raise NotImplementedError("write your pallas kernel here")


# speedup vs baseline: 6.1654x; 6.1654x over previous
import functools

import jax
import jax.numpy as jnp
from jax import lax
from jax.experimental import pallas as pl
from jax.experimental.pallas import tpu as pltpu
from jax.experimental.pallas import tpu_sc as plsc

N = 10000
E = 320000
DINP, DHID, DOUT = 128, 256, 128

NC, NS, L = 2, 16, 16
NW = NC * NS
NPAD = 10240
RPT = NPAD // NS
EPW = E // NW
EPT = E // NS
K = 125
NCH = EPT // K
G = 20
NGRP = NCH // G
BN = 640
NBLK = NPAD // BN

_mesh = plsc.VectorSubcoreMesh(
    core_axis_name="c", subcore_axis_name="s", num_cores=NC, num_subcores=NS
)


def _make_scatter_kernel(ch, nrows=NPAD):
    rpt = nrows // NS

    @functools.partial(
        pl.kernel,
        out_type=jax.ShapeDtypeStruct((NC, nrows, ch), jnp.float32),
        mesh=_mesh,
        scratch_types=[
            pltpu.VMEM((G, K), jnp.int32),
            pltpu.VMEM((G, K), jnp.int32),
            pltpu.VMEM((K, ch), jnp.float32),
            pltpu.VMEM_SHARED((nrows, ch), jnp.float32),
            pltpu.SemaphoreType.DMA,
        ],
        compiler_params=pltpu.CompilerParams(use_tc_tiling_on_sc=False),
    )
    def scatter_kernel(xs_hbm, row_hbm, col_hbm, zero_hbm, agg_hbm,
                       row_v, col_v, buf, acc, gsem):
        c = lax.axis_index("c")
        s = lax.axis_index("s")

        pltpu.sync_copy(zero_hbm.at[pl.ds(s * rpt, rpt)],
                        acc.at[pl.ds(s * rpt, rpt)])
        plsc.subcore_barrier()

        def run(ci):
            xs = xs_hbm.at[ci]

            def group(gi, _):
                pltpu.sync_copy(row_hbm.at[s * NGRP + gi], row_v)
                pltpu.sync_copy(col_hbm.at[s * NGRP + gi], col_v)

                def chunk(j, _):
                    pltpu.async_copy(xs.at[row_v.at[j]], buf, gsem).wait()
                    pltpu.sync_copy(buf, acc.at[col_v.at[j]], add=True)
                    return 0

                lax.fori_loop(0, G, chunk, 0)
                return 0

            lax.fori_loop(0, NGRP, group, 0)

        @pl.when(c == 0)
        def _():
            run(0)

        @pl.when(c == 1)
        def _():
            run(1)

        plsc.subcore_barrier()

        @pl.when(c == 0)
        def _():
            pltpu.sync_copy(acc.at[pl.ds(s * rpt, rpt)],
                            agg_hbm.at[0].at[pl.ds(s * rpt, rpt)])

        @pl.when(c == 1)
        def _():
            pltpu.sync_copy(acc.at[pl.ds(s * rpt, rpt)],
                            agg_hbm.at[1].at[pl.ds(s * rpt, rpt)])

    return scatter_kernel


_scatter128 = _make_scatter_kernel(DHID // NC)
_scatter64 = _make_scatter_kernel(DOUT // NC)
_scatter_deg = _make_scatter_kernel(L, nrows=NPAD // L)


def _b_body(x_ref, w1_ref, deg_ref, xs_ref, dinv_ref):
    deg = deg_ref[:, 0]
    dinv = jnp.where(deg > 0, lax.rsqrt(jnp.maximum(deg, 1e-12)), 0.0)
    xw = jnp.dot(x_ref[...], w1_ref[...], preferred_element_type=jnp.float32)
    xs = xw * dinv[:, None]
    xs_ref[0] = xs[:, : DHID // 2]
    xs_ref[1] = xs[:, DHID // 2:]
    dinv_ref[...] = dinv[:, None]


def _d_body(agg_ref, dinv_ref, b1_ref, w2a_ref, w2b_ref, xs2_ref):
    dinv = dinv_ref[...]
    h_a = jnp.maximum(agg_ref[0] * dinv + b1_ref[0], 0.0)
    h_b = jnp.maximum(agg_ref[1] * dinv + b1_ref[1], 0.0)
    hw = jnp.dot(h_a, w2a_ref[...], preferred_element_type=jnp.float32)
    hw = hw + jnp.dot(h_b, w2b_ref[...], preferred_element_type=jnp.float32)
    xs2 = hw * dinv
    xs2_ref[0] = xs2[:, : DOUT // 2]
    xs2_ref[1] = xs2[:, DOUT // 2:]


def _e_body(agg2_ref, dinv_ref, b2_ref, out_ref):
    dinv = dinv_ref[...]
    out_ref[:, : DOUT // 2] = jnp.maximum(agg2_ref[0] * dinv + b2_ref[0], 0.0)
    out_ref[:, DOUT // 2:] = jnp.maximum(agg2_ref[1] * dinv + b2_ref[1], 0.0)


def kernel(x, edge_index, W1, b1, W2, b2):
    ei = edge_index.astype(jnp.int32)
    row = ei[0]
    col = ei[1]
    row_c = row.reshape(NS * NGRP, G, K)
    col_c = col.reshape(NS * NGRP, G, K)
    col_hi = (col // L).reshape(NS * NGRP, G, K)
    col_lo = (col % L).reshape(NS * NGRP, G, K)
    xpad = jnp.pad(x, ((0, NPAD - N), (0, 0)))
    zero128 = jnp.zeros((NPAD, DHID // 2), jnp.float32)
    zero64 = jnp.zeros((NPAD, DOUT // 2), jnp.float32)
    zero_deg = jnp.zeros((NPAD // L, L), jnp.float32)
    eye16 = jnp.broadcast_to(jnp.eye(L, dtype=jnp.float32), (NC, L, L))

    degp = _scatter_deg(eye16, col_lo, col_hi, zero_deg)
    deg = degp[0].reshape(NPAD, 1)

    xs1, dinv = pl.pallas_call(
        _b_body,
        grid=(NBLK,),
        in_specs=[
            pl.BlockSpec((BN, DINP), lambda i: (i, 0)),
            pl.BlockSpec((DINP, DHID), lambda i: (0, 0)),
            pl.BlockSpec((BN, 1), lambda i: (i, 0)),
        ],
        out_specs=[
            pl.BlockSpec((NC, BN, DHID // 2), lambda i: (0, i, 0)),
            pl.BlockSpec((BN, 1), lambda i: (i, 0)),
        ],
        out_shape=[
            jax.ShapeDtypeStruct((NC, NPAD, DHID // 2), jnp.float32),
            jax.ShapeDtypeStruct((NPAD, 1), jnp.float32),
        ],
    )(xpad, W1, deg)

    agg1 = _scatter128(xs1, row_c, col_c, zero128)

    xs2 = pl.pallas_call(
        _d_body,
        grid=(NBLK,),
        in_specs=[
            pl.BlockSpec((NC, BN, DHID // 2), lambda i: (0, i, 0)),
            pl.BlockSpec((BN, 1), lambda i: (i, 0)),
            pl.BlockSpec((NC, 1, DHID // 2), lambda i: (0, 0, 0)),
            pl.BlockSpec((DHID // 2, DOUT), lambda i: (0, 0)),
            pl.BlockSpec((DHID // 2, DOUT), lambda i: (0, 0)),
        ],
        out_specs=pl.BlockSpec((NC, BN, DOUT // 2), lambda i: (0, i, 0)),
        out_shape=jax.ShapeDtypeStruct((NC, NPAD, DOUT // 2), jnp.float32),
    )(agg1, dinv, b1.reshape(NC, 1, DHID // 2), W2[: DHID // 2], W2[DHID // 2:])

    agg2 = _scatter64(xs2, row_c, col_c, zero64)

    outp = pl.pallas_call(
        _e_body,
        grid=(NBLK,),
        in_specs=[
            pl.BlockSpec((NC, BN, DOUT // 2), lambda i: (0, i, 0)),
            pl.BlockSpec((BN, 1), lambda i: (i, 0)),
            pl.BlockSpec((NC, 1, DOUT // 2), lambda i: (0, 0, 0)),
        ],
        out_specs=pl.BlockSpec((BN, DOUT), lambda i: (i, 0)),
        out_shape=jax.ShapeDtypeStruct((NPAD, DOUT), jnp.float32),
    )(agg2, dinv, b2.reshape(NC, 1, DOUT // 2))

    return outp[:N]


# speedup vs baseline: 10.3761x; 1.6830x over previous
import functools

import jax
import jax.numpy as jnp
from jax import lax
from jax.experimental import pallas as pl
from jax.experimental.pallas import tpu as pltpu
from jax.experimental.pallas import tpu_sc as plsc

N = 10000
E = 320000
DINP, DHID, DOUT = 128, 256, 128

NC, NS, L = 2, 16, 16
NW = NC * NS
NPAD = 10240
RPT = NPAD // NS
EPW = E // NW
EPT = E // NS
K = 125
NCH = EPT // K
G = 20
NGRP = NCH // G
BN = 640
NBLK = NPAD // BN

_mesh = plsc.VectorSubcoreMesh(
    core_axis_name="c", subcore_axis_name="s", num_cores=NC, num_subcores=NS
)


def _make_scatter_kernel(ch, nrows=NPAD, split_edges=False):
    rpt = nrows // NS
    ngrp_c = NGRP // 2 if split_edges else NGRP

    @functools.partial(
        pl.kernel,
        out_type=jax.ShapeDtypeStruct((NC, nrows, ch), jnp.float32),
        mesh=_mesh,
        scratch_types=[
            pltpu.VMEM((G, K), jnp.int32),
            pltpu.VMEM((G, K), jnp.int32),
            pltpu.VMEM((2, K, ch), jnp.float32),
            pltpu.VMEM_SHARED((nrows, ch), jnp.float32),
            pltpu.SemaphoreType.DMA,
            pltpu.SemaphoreType.DMA,
            pltpu.SemaphoreType.DMA,
            pltpu.SemaphoreType.DMA,
        ],
        compiler_params=pltpu.CompilerParams(use_tc_tiling_on_sc=False),
    )
    def scatter_kernel(xs_hbm, row_hbm, col_hbm, zero_hbm, agg_hbm,
                       row_v, col_v, buf, acc, gsem0, gsem1, ssem0, ssem1):
        c = lax.axis_index("c")
        s = lax.axis_index("s")
        gsem = (gsem0, gsem1)
        ssem = (ssem0, ssem1)

        pltpu.sync_copy(zero_hbm.at[pl.ds(s * rpt, rpt)],
                        acc.at[pl.ds(s * rpt, rpt)])
        plsc.subcore_barrier()

        def run(ci):
            xs = xs_hbm.at[ci]
            base = ci * ngrp_c if split_edges else 0

            def group(gi, _):
                pltpu.sync_copy(row_hbm.at[s * NGRP + base + gi], row_v)
                pltpu.sync_copy(col_hbm.at[s * NGRP + base + gi], col_v)
                gd, sd = {}, {}

                def g_issue(j):
                    b = j % 2
                    gd[j] = pltpu.async_copy(
                        xs.at[row_v.at[j]], buf.at[b], gsem[b])

                def s_issue(j):
                    b = j % 2
                    sd[j] = pltpu.async_copy(
                        buf.at[b], acc.at[col_v.at[j]], ssem[b], add=True)

                g_issue(0)
                for j in range(1, G):
                    if j >= 2:
                        sd[j - 2].wait()
                    g_issue(j)
                    gd[j - 1].wait()
                    s_issue(j - 1)
                gd[G - 1].wait()
                s_issue(G - 1)
                sd[G - 2].wait()
                sd[G - 1].wait()
                return 0

            lax.fori_loop(0, ngrp_c, group, 0)

        @pl.when(c == 0)
        def _():
            run(0)

        @pl.when(c == 1)
        def _():
            run(1)

        plsc.subcore_barrier()

        @pl.when(c == 0)
        def _():
            pltpu.sync_copy(acc.at[pl.ds(s * rpt, rpt)],
                            agg_hbm.at[0].at[pl.ds(s * rpt, rpt)])

        @pl.when(c == 1)
        def _():
            pltpu.sync_copy(acc.at[pl.ds(s * rpt, rpt)],
                            agg_hbm.at[1].at[pl.ds(s * rpt, rpt)])

    return scatter_kernel


_scatter128 = _make_scatter_kernel(DHID // NC)
_scatter64 = _make_scatter_kernel(DOUT // NC)
_scatter_deg = _make_scatter_kernel(L, nrows=NPAD // L, split_edges=True)


def _b_body(x_ref, w1_ref, degp_ref, xs_ref, dinv_ref):
    deg = degp_ref[0, :, 0] + degp_ref[1, :, 0]
    dinv = jnp.where(deg > 0, lax.rsqrt(jnp.maximum(deg, 1e-12)), 0.0)
    xw = jnp.dot(x_ref[...], w1_ref[...], preferred_element_type=jnp.float32)
    xs = xw * dinv[:, None]
    xs_ref[0] = xs[:, : DHID // 2]
    xs_ref[1] = xs[:, DHID // 2:]
    dinv_ref[...] = dinv[:, None]


def _d_body(agg_ref, dinv_ref, b1_ref, w2a_ref, w2b_ref, xs2_ref):
    dinv = dinv_ref[...]
    h_a = jnp.maximum(agg_ref[0] * dinv + b1_ref[0], 0.0)
    h_b = jnp.maximum(agg_ref[1] * dinv + b1_ref[1], 0.0)
    hw = jnp.dot(h_a, w2a_ref[...], preferred_element_type=jnp.float32)
    hw = hw + jnp.dot(h_b, w2b_ref[...], preferred_element_type=jnp.float32)
    xs2 = hw * dinv
    xs2_ref[0] = xs2[:, : DOUT // 2]
    xs2_ref[1] = xs2[:, DOUT // 2:]


def _e_body(agg2_ref, dinv_ref, b2_ref, out_ref):
    dinv = dinv_ref[...]
    out_ref[:, : DOUT // 2] = jnp.maximum(agg2_ref[0] * dinv + b2_ref[0], 0.0)
    out_ref[:, DOUT // 2:] = jnp.maximum(agg2_ref[1] * dinv + b2_ref[1], 0.0)


def kernel(x, edge_index, W1, b1, W2, b2):
    ei = edge_index.astype(jnp.int32)
    row = ei[0]
    col = ei[1]
    row_c = row.reshape(NS * NGRP, G, K)
    col_c = col.reshape(NS * NGRP, G, K)
    col_hi = (col // L).reshape(NS * NGRP, G, K)
    col_lo = (col % L).reshape(NS * NGRP, G, K)
    xpad = jnp.pad(x, ((0, NPAD - N), (0, 0)))
    zero128 = jnp.zeros((NPAD, DHID // 2), jnp.float32)
    zero64 = jnp.zeros((NPAD, DOUT // 2), jnp.float32)
    zero_deg = jnp.zeros((NPAD // L, L), jnp.float32)
    eye16 = jnp.broadcast_to(jnp.eye(L, dtype=jnp.float32), (NC, L, L))

    degp = _scatter_deg(eye16, col_lo, col_hi, zero_deg)
    deg = degp.reshape(NC, NPAD, 1)

    xs1, dinv = pl.pallas_call(
        _b_body,
        grid=(NBLK,),
        in_specs=[
            pl.BlockSpec((BN, DINP), lambda i: (i, 0)),
            pl.BlockSpec((DINP, DHID), lambda i: (0, 0)),
            pl.BlockSpec((NC, BN, 1), lambda i: (0, i, 0)),
        ],
        out_specs=[
            pl.BlockSpec((NC, BN, DHID // 2), lambda i: (0, i, 0)),
            pl.BlockSpec((BN, 1), lambda i: (i, 0)),
        ],
        out_shape=[
            jax.ShapeDtypeStruct((NC, NPAD, DHID // 2), jnp.float32),
            jax.ShapeDtypeStruct((NPAD, 1), jnp.float32),
        ],
    )(xpad, W1, deg)

    agg1 = _scatter128(xs1, row_c, col_c, zero128)

    xs2 = pl.pallas_call(
        _d_body,
        grid=(NBLK,),
        in_specs=[
            pl.BlockSpec((NC, BN, DHID // 2), lambda i: (0, i, 0)),
            pl.BlockSpec((BN, 1), lambda i: (i, 0)),
            pl.BlockSpec((NC, 1, DHID // 2), lambda i: (0, 0, 0)),
            pl.BlockSpec((DHID // 2, DOUT), lambda i: (0, 0)),
            pl.BlockSpec((DHID // 2, DOUT), lambda i: (0, 0)),
        ],
        out_specs=pl.BlockSpec((NC, BN, DOUT // 2), lambda i: (0, i, 0)),
        out_shape=jax.ShapeDtypeStruct((NC, NPAD, DOUT // 2), jnp.float32),
    )(agg1, dinv, b1.reshape(NC, 1, DHID // 2), W2[: DHID // 2], W2[DHID // 2:])

    agg2 = _scatter64(xs2, row_c, col_c, zero64)

    outp = pl.pallas_call(
        _e_body,
        grid=(NBLK,),
        in_specs=[
            pl.BlockSpec((NC, BN, DOUT // 2), lambda i: (0, i, 0)),
            pl.BlockSpec((BN, 1), lambda i: (i, 0)),
            pl.BlockSpec((NC, 1, DOUT // 2), lambda i: (0, 0, 0)),
        ],
        out_specs=pl.BlockSpec((BN, DOUT), lambda i: (i, 0)),
        out_shape=jax.ShapeDtypeStruct((NPAD, DOUT), jnp.float32),
    )(agg2, dinv, b2.reshape(NC, 1, DOUT // 2))

    return outp[:N]
